# baseline (speedup 1.0000x reference)
import functools

import jax
import jax.numpy as jnp
from jax import lax
from jax.experimental import pallas as pl
from jax.experimental.pallas import tpu as pltpu
from jax.experimental.pallas import tpu_sc as plsc

_L, _K, _D = 32, 1024, 2048
_LAMBDA = 0.1
_CC_LOW = 768
_CC_HIGH = 1024
_TOTAL_UNITS = 16 * _CC_LOW + 16 * _CC_HIGH

_SC_L = 8
_NTILES = 32
_GROUPS_PER_LAYER = _CC_LOW // 16
_GROUPS_PER_TILE = _SC_L * _GROUPS_PER_LAYER // _NTILES


@functools.partial(
    pl.kernel,
    mesh=plsc.VectorSubcoreMesh(core_axis_name="c", subcore_axis_name="s"),
    out_type=jax.ShapeDtypeStruct((_NTILES, 16), jnp.float32),
    scratch_types=[
        pltpu.VMEM((16 * _D,), jnp.float32),
        pltpu.VMEM((16,), jnp.float32),
    ],
    compiler_params=pltpu.CompilerParams(needs_layout_passes=False),
)
def _sc_low_layers(r_hbm, out_hbm, buf, sbuf):
    wid = lax.axis_index("s") * 2 + lax.axis_index("c")
    lanes = lax.broadcasted_iota(jnp.int32, (16,), 0)
    lane_off = lanes * _D

    def group_body(t, s_acc):
        g = wid + _NTILES * t
        layer = g // _GROUPS_PER_LAYER
        k0 = (g % _GROUPS_PER_LAYER) * 16
        row0 = layer * _K + k0
        pltpu.sync_copy(r_hbm.at[pl.ds(row0 * _D, 16 * _D)], buf)

        def col_body(j, acc):
            cols = lane_off + j * 16
            for c in range(16):
                acc = acc + plsc.load_gather(buf, [cols + c])
            return acc

        row_sum = lax.fori_loop(
            0, _D // 16, col_body, jnp.zeros((16,), jnp.float32)
        )
        r_dk = plsc.load_gather(buf, [lane_off + (lanes + k0)])
        r_minus = (row_sum - r_dk) * jnp.float32(1.0 / (_D - 1))
        a = jnp.abs(r_dk)
        b = jnp.abs(r_minus)
        return s_acc + (a - b) / (a + b + jnp.float32(1e-6))

    s_acc = lax.fori_loop(
        0, _GROUPS_PER_TILE, group_body, jnp.zeros((16,), jnp.float32)
    )
    sbuf[...] = s_acc
    pltpu.sync_copy(sbuf, out_hbm.at[wid])


def _body(x, kk):
    row_sum = jnp.sum(x, axis=1)
    col = jax.lax.broadcasted_iota(jnp.int32, x.shape, 1)
    r_dk = jnp.sum(jnp.where(col == kk[:, None], x, 0.0), axis=1)
    r_minus = (row_sum - r_dk) * jnp.float32(1.0 / (_D - 1))
    a = jnp.abs(r_dk)
    b = jnp.abs(r_minus)
    return jnp.sum((a - b) / (a + b + jnp.float32(1e-6)))


def _low_kernel(x_ref, out_ref):
    @pl.when(pl.program_id(0) == 0)
    def _init():
        out_ref[0, 0] = jnp.float32(0.0)

    r = jax.lax.broadcasted_iota(jnp.int32, (2 * _CC_LOW,), 0)
    kk = jnp.where(r >= _CC_LOW, r - _CC_LOW, r)
    x = x_ref[...].reshape(2 * _CC_LOW, _D)
    out_ref[0, 0] += _body(x, kk)


def _high_kernel(part_ref, x_ref, out_ref):
    @pl.when(pl.program_id(0) == 0)
    def _init():
        out_ref[0, 0] = part_ref[0, 0]

    kk = jnp.bitwise_and(
        jax.lax.broadcasted_iota(jnp.int32, (2 * _CC_HIGH,), 0), _K - 1
    )
    x = x_ref[...].reshape(2 * _CC_HIGH, _D)
    out_ref[0, 0] += _body(x, kk)


def _combine_kernel(tc_ref, sc_ref, out_ref):
    out_ref[0, 0] = tc_ref[0, 0] + jnp.sum(sc_ref[...])


def kernel(R):
    sc_part = _sc_low_layers(R.reshape(_L * _K * _D))
    part = pl.pallas_call(
        _low_kernel,
        grid=((16 - _SC_L) // 2,),
        in_specs=[
            pl.BlockSpec((2, _CC_LOW, _D), lambda l: (l + _SC_L // 2, 0, 0))
        ],
        out_specs=pl.BlockSpec((1, 1), lambda l: (0, 0),
                               memory_space=pltpu.SMEM),
        out_shape=jax.ShapeDtypeStruct((1, 1), jnp.float32),
    )(R)
    total_tc = pl.pallas_call(
        _high_kernel,
        grid=(8,),
        in_specs=[
            pl.BlockSpec(memory_space=pltpu.SMEM),
            pl.BlockSpec((2, _CC_HIGH, _D), lambda l: (l + 8, 0, 0)),
        ],
        out_specs=pl.BlockSpec((1, 1), lambda l: (0, 0),
                               memory_space=pltpu.SMEM),
        out_shape=jax.ShapeDtypeStruct((1, 1), jnp.float32),
    )(part, R)
    total = pl.pallas_call(
        _combine_kernel,
        in_specs=[
            pl.BlockSpec(memory_space=pltpu.SMEM),
            pl.BlockSpec(memory_space=pltpu.VMEM),
        ],
        out_specs=pl.BlockSpec(memory_space=pltpu.SMEM),
        out_shape=jax.ShapeDtypeStruct((1, 1), jnp.float32),
    )(total_tc, sc_part)
    return total[0, 0] * jnp.float32(-_LAMBDA / _TOTAL_UNITS)


# speedup vs baseline: 4.0497x; 4.0497x over previous
import functools

import jax
import jax.numpy as jnp
from jax import lax
from jax.experimental import pallas as pl
from jax.experimental.pallas import tpu as pltpu
from jax.experimental.pallas import tpu_sc as plsc

_L, _K, _D = 32, 1024, 2048
_LAMBDA = 0.1
_CC_LOW = 768
_CC_HIGH = 1024
_TOTAL_UNITS = 16 * _CC_LOW + 16 * _CC_HIGH

_SC_L = 2
_NTILES = 32
_GROUPS_PER_LAYER = _CC_LOW // 16
_GROUPS_PER_TILE = _SC_L * _GROUPS_PER_LAYER // _NTILES


@functools.partial(
    pl.kernel,
    mesh=plsc.VectorSubcoreMesh(core_axis_name="c", subcore_axis_name="s"),
    out_type=jax.ShapeDtypeStruct((_NTILES, 16), jnp.float32),
    scratch_types=[
        pltpu.VMEM((16, _D), jnp.float32),
        pltpu.VMEM((16,), jnp.float32),
    ],
    compiler_params=pltpu.CompilerParams(needs_layout_passes=False),
)
def _sc_low_layers(r_hbm, out_hbm, buf, sbuf):
    wid = lax.axis_index("s") * 2 + lax.axis_index("c")
    lanes = lax.broadcasted_iota(jnp.int32, (16,), 0)

    def group_body(t, s_acc):
        g = wid + _NTILES * t
        layer = g // _GROUPS_PER_LAYER
        k0 = (g % _GROUPS_PER_LAYER) * 16
        row0 = layer * _K + k0
        pltpu.sync_copy(r_hbm.at[pl.ds(row0, 16)], buf)

        def col_body(j, acc):
            cols = jnp.zeros((16,), jnp.int32) + j * 16
            for c in range(16):
                acc = acc + plsc.load_gather(buf, [lanes, cols + c])
            return acc

        row_sum = lax.fori_loop(
            0, _D // 16, col_body, jnp.zeros((16,), jnp.float32)
        )
        r_dk = plsc.load_gather(buf, [lanes, lanes + k0])
        r_minus = (row_sum - r_dk) * jnp.float32(1.0 / (_D - 1))
        a = jnp.abs(r_dk)
        b = jnp.abs(r_minus)
        return s_acc + (a - b) / (a + b + jnp.float32(1e-6))

    s_acc = lax.fori_loop(
        0, _GROUPS_PER_TILE, group_body, jnp.zeros((16,), jnp.float32)
    )
    sbuf[...] = s_acc
    pltpu.sync_copy(sbuf, out_hbm.at[wid])


def _body(x, kk):
    row_sum = jnp.sum(x, axis=1)
    col = jax.lax.broadcasted_iota(jnp.int32, x.shape, 1)
    r_dk = jnp.sum(jnp.where(col == kk[:, None], x, 0.0), axis=1)
    r_minus = (row_sum - r_dk) * jnp.float32(1.0 / (_D - 1))
    a = jnp.abs(r_dk)
    b = jnp.abs(r_minus)
    return jnp.sum((a - b) / (a + b + jnp.float32(1e-6)))


def _low_kernel(x_ref, out_ref):
    @pl.when(pl.program_id(0) == 0)
    def _init():
        out_ref[0, 0] = jnp.float32(0.0)

    r = jax.lax.broadcasted_iota(jnp.int32, (2 * _CC_LOW,), 0)
    kk = jnp.where(r >= _CC_LOW, r - _CC_LOW, r)
    x = x_ref[...].reshape(2 * _CC_LOW, _D)
    out_ref[0, 0] += _body(x, kk)


def _high_kernel(part_ref, x_ref, out_ref):
    @pl.when(pl.program_id(0) == 0)
    def _init():
        out_ref[0, 0] = part_ref[0, 0]

    kk = jnp.bitwise_and(
        jax.lax.broadcasted_iota(jnp.int32, (2 * _CC_HIGH,), 0), _K - 1
    )
    x = x_ref[...].reshape(2 * _CC_HIGH, _D)
    out_ref[0, 0] += _body(x, kk)


def _combine_kernel(tc_ref, sc_ref, out_ref):
    out_ref[0, 0] = tc_ref[0, 0] + jnp.sum(sc_ref[...])


def kernel(R):
    sc_part = _sc_low_layers(R.reshape(_L * _K, _D))
    part = pl.pallas_call(
        _low_kernel,
        grid=((16 - _SC_L) // 2,),
        in_specs=[
            pl.BlockSpec((2, _CC_LOW, _D), lambda l: (l + _SC_L // 2, 0, 0))
        ],
        out_specs=pl.BlockSpec((1, 1), lambda l: (0, 0),
                               memory_space=pltpu.SMEM),
        out_shape=jax.ShapeDtypeStruct((1, 1), jnp.float32),
    )(R)
    total_tc = pl.pallas_call(
        _high_kernel,
        grid=(8,),
        in_specs=[
            pl.BlockSpec(memory_space=pltpu.SMEM),
            pl.BlockSpec((2, _CC_HIGH, _D), lambda l: (l + 8, 0, 0)),
        ],
        out_specs=pl.BlockSpec((1, 1), lambda l: (0, 0),
                               memory_space=pltpu.SMEM),
        out_shape=jax.ShapeDtypeStruct((1, 1), jnp.float32),
    )(part, R)
    total = pl.pallas_call(
        _combine_kernel,
        in_specs=[
            pl.BlockSpec(memory_space=pltpu.SMEM),
            pl.BlockSpec(memory_space=pltpu.VMEM),
        ],
        out_specs=pl.BlockSpec(memory_space=pltpu.SMEM),
        out_shape=jax.ShapeDtypeStruct((1, 1), jnp.float32),
    )(total_tc, sc_part)
    return total[0, 0] * jnp.float32(-_LAMBDA / _TOTAL_UNITS)


# speedup vs baseline: 5.0082x; 1.2367x over previous
import jax
import jax.numpy as jnp
from jax.experimental import pallas as pl
from jax.experimental.pallas import tpu as pltpu

_L, _K, _D = 32, 1024, 2048
_LAMBDA = 0.1
_CC_LOW = 768
_CC_HIGH = 1024
_TOTAL_UNITS = 16 * _CC_LOW + 16 * _CC_HIGH


def _body(x, kk):
    row_sum = jnp.sum(x, axis=1)
    col = jax.lax.broadcasted_iota(jnp.int32, x.shape, 1)
    r_dk = jnp.sum(jnp.where(col == kk[:, None], x, 0.0), axis=1)
    r_minus = (row_sum - r_dk) * jnp.float32(1.0 / (_D - 1))
    a = jnp.abs(r_dk)
    b = jnp.abs(r_minus)
    return jnp.sum((a - b) / (a + b + jnp.float32(1e-6)))


def _low_kernel(x_ref, out_ref):
    @pl.when(pl.program_id(0) == 0)
    def _init():
        out_ref[0, 0] = jnp.float32(0.0)

    r = jax.lax.broadcasted_iota(jnp.int32, (2 * _CC_LOW,), 0)
    kk = jnp.where(r >= _CC_LOW, r - _CC_LOW, r)
    x = x_ref[...].reshape(2 * _CC_LOW, _D)
    out_ref[0, 0] += _body(x, kk)


def _high_kernel(part_ref, x_ref, out_ref):
    @pl.when(pl.program_id(0) == 0)
    def _init():
        out_ref[0, 0] = part_ref[0, 0]

    kk = jnp.bitwise_and(
        jax.lax.broadcasted_iota(jnp.int32, (2 * _CC_HIGH,), 0), _K - 1
    )
    x = x_ref[...].reshape(2 * _CC_HIGH, _D)
    out_ref[0, 0] += _body(x, kk)


def kernel(R):
    part = pl.pallas_call(
        _low_kernel,
        grid=(8,),
        in_specs=[pl.BlockSpec((2, _CC_LOW, _D), lambda l: (l, 0, 0))],
        out_specs=pl.BlockSpec((1, 1), lambda l: (0, 0),
                               memory_space=pltpu.SMEM),
        out_shape=jax.ShapeDtypeStruct((1, 1), jnp.float32),
    )(R)
    total = pl.pallas_call(
        _high_kernel,
        grid=(8,),
        in_specs=[
            pl.BlockSpec(memory_space=pltpu.SMEM),
            pl.BlockSpec((2, _CC_HIGH, _D), lambda l: (l + 8, 0, 0)),
        ],
        out_specs=pl.BlockSpec((1, 1), lambda l: (0, 0),
                               memory_space=pltpu.SMEM),
        out_shape=jax.ShapeDtypeStruct((1, 1), jnp.float32),
    )(part, R)
    return total[0, 0] * jnp.float32(-_LAMBDA / _TOTAL_UNITS)
